# Initial kernel scaffold; baseline (speedup 1.0000x reference)
import functools

import jax
import jax.numpy as jnp
from jax import lax
from jax.experimental import pallas as pl
from jax.experimental.pallas import tpu as pltpu
from jax.experimental.pallas import tpu_sc as plsc

N = 10000
E = 320000
H = 128

NC = 2
NS = 16
NW = NC * NS
CHUNK = 128
CPW = 80
EPW = CHUNK * CPW
E_PAD = NW * EPW
N_PAD = 10240
RPT = N_PAD // NS

_MESH = plsc.VectorSubcoreMesh(core_axis_name="c", subcore_axis_name="s")


@functools.partial(
    pl.kernel,
    out_type=jax.ShapeDtypeStruct((NC, N_PAD), jnp.float32),
    mesh=_MESH,
    scratch_types=[
        pltpu.VMEM((CHUNK,), jnp.int32),
        pltpu.VMEM((CHUNK,), jnp.float32),
        pltpu.VMEM_SHARED((N_PAD,), jnp.float32),
    ],
)
def _deg_kernel(c_hbm, ew_hbm, out_hbm, c_v, ew_v, deg_sh):
    ci = lax.axis_index("c")
    si = lax.axis_index("s")
    wid = ci * NS + si

    z = jnp.zeros((16,), jnp.float32)
    for k in range(CHUNK // 16):
        ew_v[pl.ds(k * 16, 16)] = z
    for j in range(RPT // CHUNK):
        pltpu.sync_copy(ew_v, deg_sh.at[pl.ds(si * RPT + j * CHUNK, CHUNK)])
    plsc.subcore_barrier()

    @pl.loop(0, CPW)
    def _chunk(k):
        base = wid * EPW + k * CHUNK
        pltpu.sync_copy(c_hbm.at[pl.ds(base, CHUNK)], c_v)
        pltpu.sync_copy(ew_hbm.at[pl.ds(base, CHUNK)], ew_v)
        pltpu.sync_copy(ew_v, deg_sh.at[c_v], add=True)

    plsc.subcore_barrier()
    for j in range(RPT // CHUNK):
        sl = pl.ds(si * RPT + j * CHUNK, CHUNK)
        pltpu.sync_copy(deg_sh.at[sl], out_hbm.at[ci, sl])


@functools.partial(
    pl.kernel,
    out_type=jax.ShapeDtypeStruct((NC, N_PAD, H), jnp.float32),
    mesh=_MESH,
    scratch_types=[
        pltpu.VMEM((CHUNK,), jnp.int32),
        pltpu.VMEM((CHUNK,), jnp.int32),
        pltpu.VMEM((CHUNK,), jnp.float32),
        pltpu.VMEM((CHUNK, H), jnp.float32),
        pltpu.VMEM_SHARED((N_PAD, H), jnp.float32),
        pltpu.SemaphoreType.DMA,
    ],
)
def _scatter_kernel(y_hbm, r_hbm, c_hbm, ew_hbm, out_hbm,
                    r_v, c_v, ew_v, rows_v, acc_sh, sem):
    ci = lax.axis_index("c")
    si = lax.axis_index("s")
    wid = ci * NS + si

    z = jnp.zeros((16,), jnp.float32)

    @pl.loop(0, CHUNK)
    def _zrow(i):
        for k in range(H // 16):
            rows_v[i, pl.ds(k * 16, 16)] = z

    for j in range(RPT // CHUNK):
        pltpu.sync_copy(rows_v, acc_sh.at[pl.ds(si * RPT + j * CHUNK, CHUNK)])
    plsc.subcore_barrier()

    @pl.loop(0, CPW)
    def _chunk(k):
        base = wid * EPW + k * CHUNK
        pltpu.sync_copy(r_hbm.at[pl.ds(base, CHUNK)], r_v)
        pltpu.sync_copy(c_hbm.at[pl.ds(base, CHUNK)], c_v)
        pltpu.sync_copy(ew_hbm.at[pl.ds(base, CHUNK)], ew_v)
        pltpu.async_copy(y_hbm.at[r_v], rows_v, sem).wait()

        @pl.loop(0, CHUNK)
        def _scale(e):
            sv = plsc.load_gather(ew_v, [jnp.full((16,), e, jnp.int32)])
            for kk in range(H // 16):
                sl = pl.ds(kk * 16, 16)
                rows_v[e, sl] = rows_v[e, sl] * sv

        pltpu.sync_copy(rows_v, acc_sh.at[c_v], add=True)

    plsc.subcore_barrier()
    for j in range(RPT // CHUNK):
        sl = pl.ds(si * RPT + j * CHUNK, CHUNK)
        pltpu.sync_copy(acc_sh.at[sl], out_hbm.at[ci, sl])


_BE = 2560


def _ew_body(attr_ref, a_ref, out_ref):
    sm = jax.nn.softmax(a_ref[...], axis=0)
    out_ref[...] = jnp.sum(attr_ref[...] * sm[:, 0][None, :], axis=1)


def _edge_weights(edge_attr, aaaaa):
    return pl.pallas_call(
        _ew_body,
        grid=(E // _BE,),
        in_specs=[
            pl.BlockSpec((_BE, 13), lambda i: (i, 0)),
            pl.BlockSpec((13, 1), lambda i: (0, 0)),
        ],
        out_specs=pl.BlockSpec((_BE,), lambda i: (i,)),
        out_shape=jax.ShapeDtypeStruct((E,), jnp.float32),
    )(edge_attr, aaaaa)


_BN = 1000


def _y0_body(degp_ref, x_ref, w_ref, dinv_ref, y_ref):
    deg = 1.0 + degp_ref[0, :] + degp_ref[1, :]
    dinv = lax.rsqrt(deg)
    dinv_ref[...] = dinv
    y_ref[...] = dinv[:, None] * jnp.dot(
        x_ref[...], w_ref[...], preferred_element_type=jnp.float32)


def _dinv_and_y1(degp, x, w1):
    return pl.pallas_call(
        _y0_body,
        grid=(N // _BN,),
        in_specs=[
            pl.BlockSpec((2, _BN), lambda i: (0, i)),
            pl.BlockSpec((_BN, H), lambda i: (i, 0)),
            pl.BlockSpec((H, H), lambda i: (0, 0)),
        ],
        out_specs=[
            pl.BlockSpec((_BN,), lambda i: (i,)),
            pl.BlockSpec((_BN, H), lambda i: (i, 0)),
        ],
        out_shape=[
            jax.ShapeDtypeStruct((N,), jnp.float32),
            jax.ShapeDtypeStruct((N, H), jnp.float32),
        ],
    )(degp, x, w1)


def _att(h, gw, gb):
    gate = jax.nn.sigmoid(
        jnp.sum(h * gw[:, 0][None, :], axis=1, keepdims=True) + gb[None, :])
    m = jnp.max(gate)
    ex = jnp.exp(gate - m)
    return jnp.sum(ex * h, axis=0, keepdims=True) / jnp.sum(ex)


def _comb_body(acc_ref, y_ref, dinv_ref, b_ref, gw_ref, gb_ref, wn_ref,
               r_ref, ynext_ref):
    a = acc_ref[0, :N, :] + acc_ref[1, :N, :] + y_ref[...]
    h = jnp.maximum(dinv_ref[...][:, None] * a + b_ref[...][None, :], 0.0)
    r_ref[...] = _att(h, gw_ref[...], gb_ref[...])
    ynext_ref[...] = dinv_ref[...][:, None] * jnp.dot(
        h, wn_ref[...], preferred_element_type=jnp.float32)


def _comb_last_body(acc_ref, y_ref, dinv_ref, b_ref, gw_ref, gb_ref, r_ref):
    a = acc_ref[0, :N, :] + acc_ref[1, :N, :] + y_ref[...]
    h = jnp.maximum(dinv_ref[...][:, None] * a + b_ref[...][None, :], 0.0)
    r_ref[...] = _att(h, gw_ref[...], gb_ref[...])


def _combine(acc, y, dinv, b, gw, gb, wn):
    return pl.pallas_call(
        _comb_body,
        out_shape=[
            jax.ShapeDtypeStruct((1, H), jnp.float32),
            jax.ShapeDtypeStruct((N, H), jnp.float32),
        ],
    )(acc, y, dinv, b, gw, gb, wn)


def _combine_last(acc, y, dinv, b, gw, gb):
    return pl.pallas_call(
        _comb_last_body,
        out_shape=jax.ShapeDtypeStruct((1, H), jnp.float32),
    )(acc, y, dinv, b, gw, gb)


def kernel(x, edge_index, edge_attr, aaaaa, W1, b1, W2, b2, W3, b3,
           g1W, g1b, g2W, g2b, g3W, g3b):
    ew = _edge_weights(edge_attr, aaaaa)

    pad = (jnp.arange(E_PAD - E, dtype=jnp.int32) * 97) % N
    rp = jnp.concatenate([edge_index[0], pad])
    cp = jnp.concatenate([edge_index[1], pad])
    ewp = jnp.concatenate([ew, jnp.zeros((E_PAD - E,), jnp.float32)])

    degp = _deg_kernel(cp, ewp)
    degp = degp[:, :N]
    dinv, y = _dinv_and_y1(degp, x, W1)

    acc = _scatter_kernel(y, rp, cp, ewp)
    r1, y = _combine(acc, y, dinv, b1, g1W, g1b, W2)

    acc = _scatter_kernel(y, rp, cp, ewp)
    r2, y = _combine(acc, y, dinv, b2, g2W, g2b, W3)

    acc = _scatter_kernel(y, rp, cp, ewp)
    r3 = _combine_last(acc, y, dinv, b3, g3W, g3b)

    return jnp.concatenate([r1, r2, r3], axis=1)


# speedup vs baseline: 8.4719x; 8.4719x over previous
import functools

import jax
import jax.numpy as jnp
from jax import lax
from jax.experimental import pallas as pl
from jax.experimental.pallas import tpu as pltpu
from jax.experimental.pallas import tpu_sc as plsc

N = 10000
E = 320000
H = 128

NC = 2
NS = 16
NW = NC * NS
CHUNK = 128
CPW = 80
EPW = CHUNK * CPW
E_PAD = NW * EPW
N_PAD = 10240
RPT = N_PAD // NS

def _mesh():
    return plsc.VectorSubcoreMesh(
        core_axis_name="c", subcore_axis_name="s",
        num_cores=NC, num_subcores=NS)


@functools.cache
def _make_deg_kernel():
    return functools.partial(
        pl.kernel,
        out_type=jax.ShapeDtypeStruct((NC, N_PAD), jnp.float32),
        mesh=_mesh(),
        scratch_types=[
            pltpu.VMEM((CHUNK,), jnp.int32),
            pltpu.VMEM((CHUNK,), jnp.float32),
            pltpu.VMEM_SHARED((N_PAD,), jnp.float32),
        ],
    )(_deg_body)


def _deg_body(c_hbm, ew_hbm, out_hbm, c_v, ew_v, deg_sh):
    ci = lax.axis_index("c")
    si = lax.axis_index("s")
    wid = ci * NS + si

    z = jnp.zeros((16,), jnp.float32)
    for k in range(CHUNK // 16):
        ew_v[pl.ds(k * 16, 16)] = z
    for j in range(RPT // CHUNK):
        pltpu.sync_copy(ew_v, deg_sh.at[pl.ds(si * RPT + j * CHUNK, CHUNK)])
    plsc.subcore_barrier()

    @pl.loop(0, CPW)
    def _chunk(k):
        base = wid * EPW + k * CHUNK
        pltpu.sync_copy(c_hbm.at[pl.ds(base, CHUNK)], c_v)
        pltpu.sync_copy(ew_hbm.at[pl.ds(base, CHUNK)], ew_v)
        pltpu.sync_copy(ew_v, deg_sh.at[c_v], add=True)

    plsc.subcore_barrier()
    for j in range(RPT // CHUNK):
        sl = pl.ds(si * RPT + j * CHUNK, CHUNK)
        pltpu.sync_copy(deg_sh.at[sl], out_hbm.at[ci, sl])


@functools.cache
def _make_scatter_kernel():
    return functools.partial(
        pl.kernel,
        out_type=jax.ShapeDtypeStruct((NC, N_PAD, H), jnp.float32),
        mesh=_mesh(),
        scratch_types=[
            pltpu.VMEM((CHUNK,), jnp.int32),
            pltpu.VMEM((CHUNK,), jnp.int32),
            pltpu.VMEM((CHUNK,), jnp.float32),
            pltpu.VMEM((CHUNK, H), jnp.float32),
            pltpu.VMEM_SHARED((N_PAD, H), jnp.float32),
            pltpu.SemaphoreType.DMA,
        ],
    )(_scatter_body)


def _scatter_body(y_hbm, r_hbm, c_hbm, ew_hbm, out_hbm,
                  r_v, c_v, ew_v, rows_v, acc_sh, sem):
    ci = lax.axis_index("c")
    si = lax.axis_index("s")
    wid = ci * NS + si

    z = jnp.zeros((16,), jnp.float32)

    @pl.loop(0, CHUNK)
    def _zrow(i):
        for k in range(H // 16):
            rows_v[i, pl.ds(k * 16, 16)] = z

    for j in range(RPT // CHUNK):
        pltpu.sync_copy(rows_v, acc_sh.at[pl.ds(si * RPT + j * CHUNK, CHUNK)])
    plsc.subcore_barrier()

    @pl.loop(0, CPW)
    def _chunk(k):
        base = wid * EPW + k * CHUNK
        pltpu.sync_copy(r_hbm.at[pl.ds(base, CHUNK)], r_v)
        pltpu.sync_copy(c_hbm.at[pl.ds(base, CHUNK)], c_v)
        pltpu.sync_copy(ew_hbm.at[pl.ds(base, CHUNK)], ew_v)
        pltpu.async_copy(y_hbm.at[r_v], rows_v, sem).wait()

        @pl.loop(0, CHUNK // 16)
        def _scale(g):
            ew_grp = ew_v[pl.ds(g * 16, 16)]
            for j in range(16):
                e = g * 16 + j
                sv = jnp.take(ew_grp, jnp.full((16,), j, jnp.int32),
                              mode="wrap")
                for kk in range(H // 16):
                    sl = pl.ds(kk * 16, 16)
                    rows_v[e, sl] = rows_v[e, sl] * sv

        pltpu.sync_copy(rows_v, acc_sh.at[c_v], add=True)

    plsc.subcore_barrier()
    for j in range(RPT // CHUNK):
        sl = pl.ds(si * RPT + j * CHUNK, CHUNK)
        pltpu.sync_copy(acc_sh.at[sl], out_hbm.at[ci, sl])


_BE = 2560


def _ew_body(attr_ref, a_ref, out_ref):
    i = pl.program_id(0)
    sm = jax.nn.softmax(a_ref[...], axis=0)
    out_ref[pl.ds(i * _BE, _BE)] = jnp.sum(
        attr_ref[...] * sm[:, 0][None, :], axis=1)


def _edge_weights(edge_attr, aaaaa):
    return pl.pallas_call(
        _ew_body,
        grid=(E // _BE,),
        in_specs=[
            pl.BlockSpec((_BE, 13), lambda i: (i, 0)),
            pl.BlockSpec((13, 1), lambda i: (0, 0)),
        ],
        out_specs=pl.BlockSpec((E,), lambda i: (0,)),
        out_shape=jax.ShapeDtypeStruct((E,), jnp.float32),
    )(edge_attr, aaaaa)


_BN = 1000


def _dinv_body(degp_ref, dinv_ref):
    dinv_ref[...] = lax.rsqrt(1.0 + degp_ref[0, :] + degp_ref[1, :])


def _y0_body(dinv_ref, x_ref, w_ref, y_ref):
    y_ref[...] = dinv_ref[...] * jnp.dot(
        x_ref[...], w_ref[...], preferred_element_type=jnp.float32)


def _dinv_and_y1(degp, x, w1):
    dinv = pl.pallas_call(
        _dinv_body,
        out_shape=jax.ShapeDtypeStruct((N,), jnp.float32),
    )(degp)
    dinv_col = dinv.reshape(N, 1)
    y = pl.pallas_call(
        _y0_body,
        grid=(N // _BN,),
        in_specs=[
            pl.BlockSpec((_BN, 1), lambda i: (i, 0)),
            pl.BlockSpec((_BN, H), lambda i: (i, 0)),
            pl.BlockSpec((H, H), lambda i: (0, 0)),
        ],
        out_specs=pl.BlockSpec((_BN, H), lambda i: (i, 0)),
        out_shape=jax.ShapeDtypeStruct((N, H), jnp.float32),
    )(dinv_col, x, w1)
    return dinv_col, y


def _att(h, gw, gb):
    gate = jax.nn.sigmoid(
        jnp.sum(h * gw[:, 0][None, :], axis=1, keepdims=True) + gb[None, :])
    m = jnp.max(gate)
    ex = jnp.exp(gate - m)
    return jnp.sum(ex * h, axis=0, keepdims=True) / jnp.sum(ex)


def _comb_body(acc_ref, y_ref, dinv_ref, b_ref, gw_ref, gb_ref, wn_ref,
               r_ref, ynext_ref):
    a = acc_ref[0, :N, :] + acc_ref[1, :N, :] + y_ref[...]
    h = jnp.maximum(dinv_ref[...] * a + b_ref[...][None, :], 0.0)
    r_ref[...] = _att(h, gw_ref[...], gb_ref[...])
    ynext_ref[...] = dinv_ref[...] * jnp.dot(
        h, wn_ref[...], preferred_element_type=jnp.float32)


def _comb_last_body(acc_ref, y_ref, dinv_ref, b_ref, gw_ref, gb_ref, r_ref):
    a = acc_ref[0, :N, :] + acc_ref[1, :N, :] + y_ref[...]
    h = jnp.maximum(dinv_ref[...] * a + b_ref[...][None, :], 0.0)
    r_ref[...] = _att(h, gw_ref[...], gb_ref[...])


def _combine(acc, y, dinv, b, gw, gb, wn):
    return pl.pallas_call(
        _comb_body,
        out_shape=[
            jax.ShapeDtypeStruct((1, H), jnp.float32),
            jax.ShapeDtypeStruct((N, H), jnp.float32),
        ],
    )(acc, y, dinv, b, gw, gb, wn)


def _combine_last(acc, y, dinv, b, gw, gb):
    return pl.pallas_call(
        _comb_last_body,
        out_shape=jax.ShapeDtypeStruct((1, H), jnp.float32),
    )(acc, y, dinv, b, gw, gb)


def kernel(x, edge_index, edge_attr, aaaaa, W1, b1, W2, b2, W3, b3,
           g1W, g1b, g2W, g2b, g3W, g3b):
    ew = _edge_weights(edge_attr, aaaaa)

    pad = (jnp.arange(E_PAD - E, dtype=jnp.int32) * 97) % N
    rp = jnp.concatenate([edge_index[0], pad])
    cp = jnp.concatenate([edge_index[1], pad])
    ewp = jnp.concatenate([ew, jnp.zeros((E_PAD - E,), jnp.float32)])

    degp = _make_deg_kernel()(cp, ewp)
    degp = degp[:, :N]
    dinv, y = _dinv_and_y1(degp, x, W1)

    scatter = _make_scatter_kernel()
    acc = scatter(y, rp, cp, ewp)
    r1, y = _combine(acc, y, dinv, b1, g1W, g1b, W2)

    acc = scatter(y, rp, cp, ewp)
    r2, y = _combine(acc, y, dinv, b2, g2W, g2b, W3)

    acc = scatter(y, rp, cp, ewp)
    r3 = _combine_last(acc, y, dinv, b3, g3W, g3b)

    return jnp.concatenate([r1, r2, r3], axis=1)


# speedup vs baseline: 14.6729x; 1.7319x over previous
import functools

import jax
import jax.numpy as jnp
from jax import lax
from jax.experimental import pallas as pl
from jax.experimental.pallas import tpu as pltpu
from jax.experimental.pallas import tpu_sc as plsc

N = 10000
E = 320000
H = 128

NC = 2
NS = 16
NW = NC * NS
CHUNK = 128
CPW = 80
EPW = CHUNK * CPW
E_PAD = NW * EPW
N_PAD = 10240
RPT = N_PAD // NS

def _mesh():
    return plsc.VectorSubcoreMesh(
        core_axis_name="c", subcore_axis_name="s",
        num_cores=NC, num_subcores=NS)


@functools.cache
def _make_deg_kernel():
    return functools.partial(
        pl.kernel,
        out_type=jax.ShapeDtypeStruct((NC, N_PAD), jnp.float32),
        mesh=_mesh(),
        scratch_types=[
            pltpu.VMEM((CHUNK,), jnp.int32),
            pltpu.VMEM((CHUNK,), jnp.float32),
            pltpu.VMEM_SHARED((N_PAD,), jnp.float32),
        ],
    )(_deg_body)


def _deg_body(c_hbm, ew_hbm, out_hbm, c_v, ew_v, deg_sh):
    ci = lax.axis_index("c")
    si = lax.axis_index("s")
    wid = ci * NS + si

    z = jnp.zeros((16,), jnp.float32)
    for k in range(CHUNK // 16):
        ew_v[pl.ds(k * 16, 16)] = z
    for j in range(RPT // CHUNK):
        pltpu.sync_copy(ew_v, deg_sh.at[pl.ds(si * RPT + j * CHUNK, CHUNK)])
    plsc.subcore_barrier()

    @pl.loop(0, CPW)
    def _chunk(k):
        base = wid * EPW + k * CHUNK
        pltpu.sync_copy(c_hbm.at[pl.ds(base, CHUNK)], c_v)
        pltpu.sync_copy(ew_hbm.at[pl.ds(base, CHUNK)], ew_v)
        pltpu.sync_copy(ew_v, deg_sh.at[c_v], add=True)

    plsc.subcore_barrier()
    for j in range(RPT // CHUNK):
        sl = pl.ds(si * RPT + j * CHUNK, CHUNK)
        pltpu.sync_copy(deg_sh.at[sl], out_hbm.at[ci, sl])


@functools.cache
def _make_scatter_kernel():
    return functools.partial(
        pl.kernel,
        out_type=jax.ShapeDtypeStruct((NC, N_PAD, H), jnp.float32),
        mesh=_mesh(),
        scratch_types=[
            pltpu.VMEM((4, CHUNK), jnp.int32),
            pltpu.VMEM((4, CHUNK), jnp.int32),
            pltpu.VMEM((4, CHUNK), jnp.float32),
            pltpu.VMEM((2, CHUNK, H), jnp.float32),
            pltpu.VMEM_SHARED((N_PAD, H), jnp.float32),
            pltpu.SemaphoreType.DMA((4,)),
            pltpu.SemaphoreType.DMA((2,)),
        ],
    )(_scatter_body)


def _scatter_body(y_hbm, r_hbm, c_hbm, ew_hbm, out_hbm,
                  r_v, c_v, ew_v, rows_v, acc_sh, sem_idx, sem_g):
    ci = lax.axis_index("c")
    si = lax.axis_index("s")
    wid = ci * NS + si
    ebase = wid * EPW

    z = jnp.zeros((16,), jnp.float32)

    @pl.loop(0, CHUNK)
    def _zrow(i):
        for k in range(H // 16):
            rows_v[0, i, pl.ds(k * 16, 16)] = z

    for j in range(RPT // CHUNK):
        pltpu.sync_copy(rows_v.at[0],
                        acc_sh.at[pl.ds(si * RPT + j * CHUNK, CHUNK)])
    plsc.subcore_barrier()

    def _start_idx(k, b):
        base = ebase + k * CHUNK
        pltpu.async_copy(r_hbm.at[pl.ds(base, CHUNK)], r_v.at[b], sem_idx.at[b])
        pltpu.async_copy(c_hbm.at[pl.ds(base, CHUNK)], c_v.at[b], sem_idx.at[b])
        pltpu.async_copy(ew_hbm.at[pl.ds(base, CHUNK)], ew_v.at[b],
                         sem_idx.at[b])

    def _wait_idx(b):
        pltpu.make_async_copy(r_hbm.at[pl.ds(0, CHUNK)], r_v.at[b],
                              sem_idx.at[b]).wait()
        pltpu.make_async_copy(c_hbm.at[pl.ds(0, CHUNK)], c_v.at[b],
                              sem_idx.at[b]).wait()
        pltpu.make_async_copy(ew_hbm.at[pl.ds(0, CHUNK)], ew_v.at[b],
                              sem_idx.at[b]).wait()

    def _start_gather(q, b):
        pltpu.async_copy(y_hbm.at[r_v.at[q]], rows_v.at[b], sem_g.at[b])

    def _wait_gather(q, b):
        pltpu.make_async_copy(y_hbm.at[r_v.at[q]], rows_v.at[b],
                              sem_g.at[b]).wait()

    _start_idx(0, 0)
    _start_idx(1, 1)
    _start_idx(2, 2)
    _wait_idx(0)
    _start_gather(0, 0)

    @pl.loop(0, CPW // 4)
    def _chunk(kk):
        for j4 in range(4):
            k = kk * 4 + j4
            q = j4
            b = j4 % 2

            @pl.when(k + 1 < CPW)
            def _():
                _wait_idx((q + 1) % 4)
                _start_gather((q + 1) % 4, 1 - b)

            @pl.when(k + 3 < CPW)
            def _():
                _start_idx(k + 3, (q + 3) % 4)

            _wait_gather(q, b)

            @pl.loop(0, CHUNK // 16)
            def _scale(g):
                ew_grp = ew_v[q, pl.ds(g * 16, 16)]
                for j in range(16):
                    e = g * 16 + j
                    sv = jnp.take(ew_grp, jnp.full((16,), j, jnp.int32),
                                  mode="wrap")
                    for kf in range(H // 16):
                        sl = pl.ds(kf * 16, 16)
                        rows_v[b, e, sl] = rows_v[b, e, sl] * sv

            pltpu.sync_copy(rows_v.at[b], acc_sh.at[c_v.at[q]], add=True)

    plsc.subcore_barrier()
    for j in range(RPT // CHUNK):
        sl = pl.ds(si * RPT + j * CHUNK, CHUNK)
        pltpu.sync_copy(acc_sh.at[sl], out_hbm.at[ci, sl])


_BE = 2560


def _ew_body(attr_ref, a_ref, out_ref):
    i = pl.program_id(0)
    sm = jax.nn.softmax(a_ref[...], axis=0)
    out_ref[pl.ds(i * _BE, _BE)] = jnp.sum(
        attr_ref[...] * sm[:, 0][None, :], axis=1)


def _edge_weights(edge_attr, aaaaa):
    return pl.pallas_call(
        _ew_body,
        grid=(E // _BE,),
        in_specs=[
            pl.BlockSpec((_BE, 13), lambda i: (i, 0)),
            pl.BlockSpec((13, 1), lambda i: (0, 0)),
        ],
        out_specs=pl.BlockSpec((E,), lambda i: (0,)),
        out_shape=jax.ShapeDtypeStruct((E,), jnp.float32),
    )(edge_attr, aaaaa)


_BN = 1000


def _dinv_body(degp_ref, dinv_ref):
    dinv_ref[...] = lax.rsqrt(1.0 + degp_ref[0, :] + degp_ref[1, :])


def _y0_body(dinv_ref, x_ref, w_ref, y_ref):
    y_ref[...] = dinv_ref[...] * jnp.dot(
        x_ref[...], w_ref[...], preferred_element_type=jnp.float32)


def _dinv_and_y1(degp, x, w1):
    dinv = pl.pallas_call(
        _dinv_body,
        out_shape=jax.ShapeDtypeStruct((N,), jnp.float32),
    )(degp)
    dinv_col = dinv.reshape(N, 1)
    y = pl.pallas_call(
        _y0_body,
        grid=(N // _BN,),
        in_specs=[
            pl.BlockSpec((_BN, 1), lambda i: (i, 0)),
            pl.BlockSpec((_BN, H), lambda i: (i, 0)),
            pl.BlockSpec((H, H), lambda i: (0, 0)),
        ],
        out_specs=pl.BlockSpec((_BN, H), lambda i: (i, 0)),
        out_shape=jax.ShapeDtypeStruct((N, H), jnp.float32),
    )(dinv_col, x, w1)
    return dinv_col, y


def _att(h, gw, gb):
    gate = jax.nn.sigmoid(
        jnp.sum(h * gw[:, 0][None, :], axis=1, keepdims=True) + gb[None, :])
    m = jnp.max(gate)
    ex = jnp.exp(gate - m)
    return jnp.sum(ex * h, axis=0, keepdims=True) / jnp.sum(ex)


def _comb_body(acc_ref, y_ref, dinv_ref, b_ref, gw_ref, gb_ref, wn_ref,
               r_ref, ynext_ref):
    a = acc_ref[0, :N, :] + acc_ref[1, :N, :] + y_ref[...]
    h = jnp.maximum(dinv_ref[...] * a + b_ref[...][None, :], 0.0)
    r_ref[...] = _att(h, gw_ref[...], gb_ref[...])
    ynext_ref[...] = dinv_ref[...] * jnp.dot(
        h, wn_ref[...], preferred_element_type=jnp.float32)


def _comb_last_body(acc_ref, y_ref, dinv_ref, b_ref, gw_ref, gb_ref, r_ref):
    a = acc_ref[0, :N, :] + acc_ref[1, :N, :] + y_ref[...]
    h = jnp.maximum(dinv_ref[...] * a + b_ref[...][None, :], 0.0)
    r_ref[...] = _att(h, gw_ref[...], gb_ref[...])


def _combine(acc, y, dinv, b, gw, gb, wn):
    return pl.pallas_call(
        _comb_body,
        out_shape=[
            jax.ShapeDtypeStruct((1, H), jnp.float32),
            jax.ShapeDtypeStruct((N, H), jnp.float32),
        ],
    )(acc, y, dinv, b, gw, gb, wn)


def _combine_last(acc, y, dinv, b, gw, gb):
    return pl.pallas_call(
        _comb_last_body,
        out_shape=jax.ShapeDtypeStruct((1, H), jnp.float32),
    )(acc, y, dinv, b, gw, gb)


def kernel(x, edge_index, edge_attr, aaaaa, W1, b1, W2, b2, W3, b3,
           g1W, g1b, g2W, g2b, g3W, g3b):
    ew = _edge_weights(edge_attr, aaaaa)

    pad = (jnp.arange(E_PAD - E, dtype=jnp.int32) * 97) % N
    rp = jnp.concatenate([edge_index[0], pad])
    cp = jnp.concatenate([edge_index[1], pad])
    ewp = jnp.concatenate([ew, jnp.zeros((E_PAD - E,), jnp.float32)])

    degp = _make_deg_kernel()(cp, ewp)
    degp = degp[:, :N]
    dinv, y = _dinv_and_y1(degp, x, W1)

    scatter = _make_scatter_kernel()
    acc = scatter(y, rp, cp, ewp)
    r1, y = _combine(acc, y, dinv, b1, g1W, g1b, W2)

    acc = scatter(y, rp, cp, ewp)
    r2, y = _combine(acc, y, dinv, b2, g2W, g2b, W3)

    acc = scatter(y, rp, cp, ewp)
    r3 = _combine_last(acc, y, dinv, b3, g3W, g3b)

    return jnp.concatenate([r1, r2, r3], axis=1)


# speedup vs baseline: 15.8896x; 1.0829x over previous
import functools

import jax
import jax.numpy as jnp
from jax import lax
from jax.experimental import pallas as pl
from jax.experimental.pallas import tpu as pltpu
from jax.experimental.pallas import tpu_sc as plsc

N = 10000
E = 320000
H = 128

NC = 2
NS = 16
NW = NC * NS
CHUNK = 64
CPW = 160
EPW = CHUNK * CPW
E_PAD = NW * EPW
N_PAD = 10240
RPT = N_PAD // NS

def _mesh():
    return plsc.VectorSubcoreMesh(
        core_axis_name="c", subcore_axis_name="s",
        num_cores=NC, num_subcores=NS)


@functools.cache
def _make_deg_kernel():
    return functools.partial(
        pl.kernel,
        out_type=jax.ShapeDtypeStruct((NC, N_PAD), jnp.float32),
        mesh=_mesh(),
        scratch_types=[
            pltpu.VMEM((8, CHUNK), jnp.int32),
            pltpu.VMEM((8, CHUNK), jnp.float32),
            pltpu.VMEM((128,), jnp.float32),
            pltpu.VMEM_SHARED((N_PAD,), jnp.float32),
            pltpu.SemaphoreType.DMA((8,)),
            pltpu.SemaphoreType.DMA((4,)),
        ],
    )(_deg_body)


def _deg_body(c_hbm, ew_hbm, out_hbm, c_v, ew_v, zbuf, deg_sh, sem_idx, sem_s):
    ci = lax.axis_index("c")
    si = lax.axis_index("s")
    wid = ci * NS + si
    ebase = wid * EPW

    z = jnp.zeros((16,), jnp.float32)
    for k in range(128 // 16):
        zbuf[pl.ds(k * 16, 16)] = z
    for j in range(RPT // 128):
        pltpu.sync_copy(zbuf, deg_sh.at[pl.ds(si * RPT + j * 128, 128)])
    plsc.subcore_barrier()

    def _start_idx(k, q):
        base = ebase + k * CHUNK
        pltpu.async_copy(c_hbm.at[pl.ds(base, CHUNK)], c_v.at[q],
                         sem_idx.at[q])
        pltpu.async_copy(ew_hbm.at[pl.ds(base, CHUNK)], ew_v.at[q],
                         sem_idx.at[q])

    def _wait_idx(q):
        pltpu.make_async_copy(c_hbm.at[pl.ds(0, CHUNK)], c_v.at[q],
                              sem_idx.at[q]).wait()
        pltpu.make_async_copy(ew_hbm.at[pl.ds(0, CHUNK)], ew_v.at[q],
                              sem_idx.at[q]).wait()

    def _scat_desc(q, sb):
        return pltpu.make_async_copy(ew_v.at[q], deg_sh.at[c_v.at[q]],
                                     sem_s.at[sb])

    for p in range(5):
        _start_idx(p, p)
    _wait_idx(0)

    @pl.loop(0, CPW // 8)
    def _chunk(kk):
        for j8 in range(8):
            k = kk * 8 + j8
            q = j8
            sb = j8 % 4

            @pl.when(k + 1 < CPW)
            def _():
                _wait_idx((q + 1) % 8)

            @pl.when(k >= 3)
            def _():
                _scat_desc((q + 5) % 8, (q + 1) % 4).wait()

            @pl.when(k + 5 < CPW)
            def _():
                _start_idx(k + 5, (q + 5) % 8)

            _scat_desc(q, sb).start(add=True)

    _scat_desc(5, 1).wait()
    _scat_desc(6, 2).wait()
    _scat_desc(7, 3).wait()

    plsc.subcore_barrier()
    for j in range(RPT // 128):
        sl = pl.ds(si * RPT + j * 128, 128)
        pltpu.sync_copy(deg_sh.at[sl], out_hbm.at[ci, sl])


@functools.cache
def _make_scatter_kernel():
    return functools.partial(
        pl.kernel,
        out_type=jax.ShapeDtypeStruct((NC, N_PAD, H), jnp.float32),
        mesh=_mesh(),
        scratch_types=[
            pltpu.VMEM((8, CHUNK), jnp.int32),
            pltpu.VMEM((8, CHUNK), jnp.int32),
            pltpu.VMEM((8, CHUNK), jnp.float32),
            pltpu.VMEM((4, CHUNK, H), jnp.float32),
            pltpu.VMEM_SHARED((N_PAD, H), jnp.float32),
            pltpu.SemaphoreType.DMA((8,)),
            pltpu.SemaphoreType.DMA((4,)),
            pltpu.SemaphoreType.DMA((4,)),
        ],
    )(_scatter_body)


def _scatter_body(y_hbm, r_hbm, c_hbm, ew_hbm, out_hbm,
                  r_v, c_v, ew_v, rows_v, acc_sh, sem_idx, sem_g, sem_s):
    ci = lax.axis_index("c")
    si = lax.axis_index("s")
    wid = ci * NS + si
    ebase = wid * EPW

    z = jnp.zeros((16,), jnp.float32)

    @pl.loop(0, CHUNK)
    def _zrow(i):
        for k in range(H // 16):
            rows_v[0, i, pl.ds(k * 16, 16)] = z

    for j in range(RPT // CHUNK):
        pltpu.sync_copy(rows_v.at[0],
                        acc_sh.at[pl.ds(si * RPT + j * CHUNK, CHUNK)])
    plsc.subcore_barrier()

    def _start_idx(k, q):
        base = ebase + k * CHUNK
        pltpu.async_copy(r_hbm.at[pl.ds(base, CHUNK)], r_v.at[q], sem_idx.at[q])
        pltpu.async_copy(c_hbm.at[pl.ds(base, CHUNK)], c_v.at[q], sem_idx.at[q])
        pltpu.async_copy(ew_hbm.at[pl.ds(base, CHUNK)], ew_v.at[q],
                         sem_idx.at[q])

    def _wait_idx(q):
        pltpu.make_async_copy(r_hbm.at[pl.ds(0, CHUNK)], r_v.at[q],
                              sem_idx.at[q]).wait()
        pltpu.make_async_copy(c_hbm.at[pl.ds(0, CHUNK)], c_v.at[q],
                              sem_idx.at[q]).wait()
        pltpu.make_async_copy(ew_hbm.at[pl.ds(0, CHUNK)], ew_v.at[q],
                              sem_idx.at[q]).wait()

    def _gather_desc(q, rb):
        return pltpu.make_async_copy(y_hbm.at[r_v.at[q]], rows_v.at[rb],
                                     sem_g.at[rb])

    def _scat_desc(q, rb):
        return pltpu.make_async_copy(rows_v.at[rb], acc_sh.at[c_v.at[q]],
                                     sem_s.at[rb])

    for p in range(5):
        _start_idx(p, p)
    _wait_idx(0)
    _gather_desc(0, 0).start()

    @pl.loop(0, CPW // 8)
    def _chunk(kk):
        for j8 in range(8):
            k = kk * 8 + j8
            q = j8
            rb = j8 % 4

            @pl.when(k + 1 < CPW)
            def _():
                _wait_idx((q + 1) % 8)

            @pl.when(k >= 3)
            def _():
                _scat_desc((q + 5) % 8, (rb + 1) % 4).wait()

            @pl.when(k + 1 < CPW)
            def _():
                _gather_desc((q + 1) % 8, (rb + 1) % 4).start()

            @pl.when(k + 5 < CPW)
            def _():
                _start_idx(k + 5, (q + 5) % 8)

            _gather_desc(q, rb).wait()

            @pl.loop(0, CHUNK // 16)
            def _scale(g):
                ew_grp = ew_v[q, pl.ds(g * 16, 16)]
                for j in range(16):
                    e = g * 16 + j
                    sv = jnp.take(ew_grp, jnp.full((16,), j, jnp.int32),
                                  mode="wrap")
                    for kf in range(H // 16):
                        sl = pl.ds(kf * 16, 16)
                        rows_v[rb, e, sl] = rows_v[rb, e, sl] * sv

            _scat_desc(q, rb).start(add=True)

    _scat_desc(5, 1).wait()
    _scat_desc(6, 2).wait()
    _scat_desc(7, 3).wait()

    plsc.subcore_barrier()
    for j in range(RPT // CHUNK):
        sl = pl.ds(si * RPT + j * CHUNK, CHUNK)
        pltpu.sync_copy(acc_sh.at[sl], out_hbm.at[ci, sl])


_BE = 2560


def _ew_body(attr_ref, a_ref, out_ref):
    i = pl.program_id(0)
    sm = jax.nn.softmax(a_ref[...], axis=0)
    out_ref[pl.ds(i * _BE, _BE)] = jnp.sum(
        attr_ref[...] * sm[:, 0][None, :], axis=1)


def _edge_weights(edge_attr, aaaaa):
    return pl.pallas_call(
        _ew_body,
        grid=(E // _BE,),
        in_specs=[
            pl.BlockSpec((_BE, 13), lambda i: (i, 0)),
            pl.BlockSpec((13, 1), lambda i: (0, 0)),
        ],
        out_specs=pl.BlockSpec((E,), lambda i: (0,)),
        out_shape=jax.ShapeDtypeStruct((E,), jnp.float32),
    )(edge_attr, aaaaa)


_BN = 1000


def _dinv_body(degp_ref, dinv_ref):
    dinv_ref[...] = lax.rsqrt(1.0 + degp_ref[0, :] + degp_ref[1, :])


def _y0_body(dinv_ref, x_ref, w_ref, y_ref):
    y_ref[...] = dinv_ref[...] * jnp.dot(
        x_ref[...], w_ref[...], preferred_element_type=jnp.float32)


def _dinv_and_y1(degp, x, w1):
    dinv = pl.pallas_call(
        _dinv_body,
        out_shape=jax.ShapeDtypeStruct((N,), jnp.float32),
    )(degp)
    dinv_col = dinv.reshape(N, 1)
    y = pl.pallas_call(
        _y0_body,
        grid=(N // _BN,),
        in_specs=[
            pl.BlockSpec((_BN, 1), lambda i: (i, 0)),
            pl.BlockSpec((_BN, H), lambda i: (i, 0)),
            pl.BlockSpec((H, H), lambda i: (0, 0)),
        ],
        out_specs=pl.BlockSpec((_BN, H), lambda i: (i, 0)),
        out_shape=jax.ShapeDtypeStruct((N, H), jnp.float32),
    )(dinv_col, x, w1)
    return dinv_col, y


def _att(h, gw, gb):
    gate = jax.nn.sigmoid(
        jnp.sum(h * gw[:, 0][None, :], axis=1, keepdims=True) + gb[None, :])
    m = jnp.max(gate)
    ex = jnp.exp(gate - m)
    return jnp.sum(ex * h, axis=0, keepdims=True) / jnp.sum(ex)


def _comb_body(acc_ref, y_ref, dinv_ref, b_ref, gw_ref, gb_ref, wn_ref,
               r_ref, ynext_ref):
    a = acc_ref[0, :N, :] + acc_ref[1, :N, :] + y_ref[...]
    h = jnp.maximum(dinv_ref[...] * a + b_ref[...][None, :], 0.0)
    r_ref[...] = _att(h, gw_ref[...], gb_ref[...])
    ynext_ref[...] = dinv_ref[...] * jnp.dot(
        h, wn_ref[...], preferred_element_type=jnp.float32)


def _comb_last_body(acc_ref, y_ref, dinv_ref, b_ref, gw_ref, gb_ref, r_ref):
    a = acc_ref[0, :N, :] + acc_ref[1, :N, :] + y_ref[...]
    h = jnp.maximum(dinv_ref[...] * a + b_ref[...][None, :], 0.0)
    r_ref[...] = _att(h, gw_ref[...], gb_ref[...])


def _combine(acc, y, dinv, b, gw, gb, wn):
    return pl.pallas_call(
        _comb_body,
        out_shape=[
            jax.ShapeDtypeStruct((1, H), jnp.float32),
            jax.ShapeDtypeStruct((N, H), jnp.float32),
        ],
    )(acc, y, dinv, b, gw, gb, wn)


def _combine_last(acc, y, dinv, b, gw, gb):
    return pl.pallas_call(
        _comb_last_body,
        out_shape=jax.ShapeDtypeStruct((1, H), jnp.float32),
    )(acc, y, dinv, b, gw, gb)


def kernel(x, edge_index, edge_attr, aaaaa, W1, b1, W2, b2, W3, b3,
           g1W, g1b, g2W, g2b, g3W, g3b):
    ew = _edge_weights(edge_attr, aaaaa)

    pad = (jnp.arange(E_PAD - E, dtype=jnp.int32) * 97) % N
    rp = jnp.concatenate([edge_index[0], pad])
    cp = jnp.concatenate([edge_index[1], pad])
    ewp = jnp.concatenate([ew, jnp.zeros((E_PAD - E,), jnp.float32)])

    degp = _make_deg_kernel()(cp, ewp)
    degp = degp[:, :N]
    dinv, y = _dinv_and_y1(degp, x, W1)

    scatter = _make_scatter_kernel()
    acc = scatter(y, rp, cp, ewp)
    r1, y = _combine(acc, y, dinv, b1, g1W, g1b, W2)

    acc = scatter(y, rp, cp, ewp)
    r2, y = _combine(acc, y, dinv, b2, g2W, g2b, W3)

    acc = scatter(y, rp, cp, ewp)
    r3 = _combine_last(acc, y, dinv, b3, g3W, g3b)

    return jnp.concatenate([r1, r2, r3], axis=1)
